# Initial kernel scaffold; baseline (speedup 1.0000x reference)
import jax
import jax.numpy as jnp
from jax.experimental import pallas as pl


def kernel(x, U_w, U_b, V_w, V_b):
    raise NotImplementedError("write your pallas kernel here")


# speedup vs baseline: 3.3779x; 3.3779x over previous
import functools

import jax
import jax.numpy as jnp
from jax.experimental import pallas as pl
from jax.experimental.pallas import tpu as pltpu

NEIGHBOURS = 8


def _gcn_body(x_ref, ut_ref, vt_ref, ub_ref, vb_ref, o_ref, *, t):
    xb = x_ref[0]

    sim = jax.lax.dot_general(
        xb, xb, (((1,), (1,)), ((), ())), preferred_element_type=jnp.float32
    )

    work = sim
    thr = jnp.zeros((t, 1), jnp.float32)
    cnt = jnp.zeros((t, 1), jnp.float32)
    neg = jnp.float32(-jnp.inf)
    for i in range(NEIGHBOURS):
        cur = jnp.max(work, axis=1, keepdims=True)
        eq = work == cur
        if i == 0:
            thr = cur
        else:
            thr = jnp.where(cnt < NEIGHBOURS, cur, thr)
        cnt = cnt + jnp.sum(eq.astype(jnp.float32), axis=1, keepdims=True)
        work = jnp.where(eq, neg, work)

    adj = (sim >= thr).astype(jnp.float32)
    deg = jnp.sum(adj, axis=1, keepdims=True)
    dinv = jax.lax.rsqrt(deg)

    vx = (
        jnp.dot(xb, vt_ref[...], preferred_element_type=jnp.float32)
        + vb_ref[...]
    )
    agg = (
        jnp.dot(adj, vx * dinv, preferred_element_type=jnp.float32) * dinv
    )
    ux = (
        jnp.dot(xb, ut_ref[...], preferred_element_type=jnp.float32)
        + ub_ref[...]
    )
    o_ref[0] = jnp.maximum(xb + agg + ux, 0.0)


@jax.jit
def kernel(x, U_w, U_b, V_w, V_b):
    b, t, j, c = x.shape
    x3 = x.reshape(b, t, j * c)
    ut = U_w.T
    vt = V_w.T
    ub = U_b.reshape(1, c)
    vb = V_b.reshape(1, c)

    out3 = pl.pallas_call(
        functools.partial(_gcn_body, t=t),
        grid=(b * j,),
        in_specs=[
            pl.BlockSpec((1, t, c), lambda g: (g // j, 0, g % j)),
            pl.BlockSpec((c, c), lambda g: (0, 0)),
            pl.BlockSpec((c, c), lambda g: (0, 0)),
            pl.BlockSpec((1, c), lambda g: (0, 0)),
            pl.BlockSpec((1, c), lambda g: (0, 0)),
        ],
        out_specs=pl.BlockSpec((1, t, c), lambda g: (g // j, 0, g % j)),
        out_shape=jax.ShapeDtypeStruct((b, t, j * c), jnp.float32),
        compiler_params=pltpu.CompilerParams(
            dimension_semantics=("arbitrary",),
        ),
    )(x3, ut, vt, ub, vb)
    return out3.reshape(b, t, j, c)


# speedup vs baseline: 3.5898x; 1.0627x over previous
import functools

import jax
import jax.numpy as jnp
from jax.experimental import pallas as pl
from jax.experimental.pallas import tpu as pltpu

NEIGHBOURS = 8
NB = 4


def _gcn_body(x_ref, ut_ref, vt_ref, ub_ref, vb_ref, o_ref, *, t, nb):
    ones = jnp.ones((t, 128), jnp.float32)
    neg = jnp.float32(-jnp.inf)
    for i in range(nb):
        xb = x_ref[i]

        sim = jax.lax.dot_general(
            xb, xb, (((1,), (1,)), ((), ())),
            preferred_element_type=jnp.float32,
        )

        work = sim
        thr = None
        for it in range(NEIGHBOURS):
            thr = jnp.max(work, axis=1, keepdims=True)
            if it < NEIGHBOURS - 1:
                work = jnp.where(work >= thr, neg, work)

        adj = (sim >= thr).astype(jnp.float32)
        deg = jnp.dot(adj, ones, preferred_element_type=jnp.float32)[:, 0:1]
        dinv = jax.lax.rsqrt(deg)

        vx = (
            jnp.dot(xb, vt_ref[...], preferred_element_type=jnp.float32)
            + vb_ref[...]
        )
        agg = (
            jnp.dot(adj, vx * dinv, preferred_element_type=jnp.float32)
            * dinv
        )
        ux = (
            jnp.dot(xb, ut_ref[...], preferred_element_type=jnp.float32)
            + ub_ref[...]
        )
        o_ref[i] = jnp.maximum(xb + agg + ux, 0.0)


@jax.jit
def kernel(x, U_w, U_b, V_w, V_b):
    b, t, j, c = x.shape
    x3 = x.reshape(b, t, j * c)
    ut = U_w.T
    vt = V_w.T
    ub = U_b.reshape(1, c)
    vb = V_b.reshape(1, c)

    out3 = pl.pallas_call(
        functools.partial(_gcn_body, t=t, nb=NB),
        grid=(b // NB, j),
        in_specs=[
            pl.BlockSpec((NB, t, c), lambda bi, ji: (bi, 0, ji)),
            pl.BlockSpec((c, c), lambda bi, ji: (0, 0)),
            pl.BlockSpec((c, c), lambda bi, ji: (0, 0)),
            pl.BlockSpec((1, c), lambda bi, ji: (0, 0)),
            pl.BlockSpec((1, c), lambda bi, ji: (0, 0)),
        ],
        out_specs=pl.BlockSpec((NB, t, c), lambda bi, ji: (bi, 0, ji)),
        out_shape=jax.ShapeDtypeStruct((b, t, j * c), jnp.float32),
        compiler_params=pltpu.CompilerParams(
            dimension_semantics=("arbitrary", "arbitrary"),
        ),
    )(x3, ut, vt, ub, vb)
    return out3.reshape(b, t, j, c)


# speedup vs baseline: 4.2540x; 1.1850x over previous
import functools

import jax
import jax.numpy as jnp
from jax.experimental import pallas as pl
from jax.experimental.pallas import tpu as pltpu

NEIGHBOURS = 8
NB = 4


def _gcn_body(x_ref, ut_ref, vt_ref, ub_ref, vb_ref, o_ref, *, t, nb):
    xs = x_ref[...]

    sim = jax.lax.dot_general(
        xs, xs, (((2,), (2,)), ((0,), (0,))),
        preferred_element_type=jnp.float32,
    )

    neg = jnp.float32(-jnp.inf)
    work = sim
    thr = None
    for it in range(NEIGHBOURS):
        thr = jnp.max(work, axis=2, keepdims=True)
        if it < NEIGHBOURS - 1:
            work = jnp.where(work >= thr, neg, work)

    adj = (sim >= thr).astype(jnp.float32)
    ones = jnp.ones((nb, t, 128), jnp.float32)
    deg = jax.lax.dot_general(
        adj, ones, (((2,), (1,)), ((0,), (0,))),
        preferred_element_type=jnp.float32,
    )[:, :, 0:1]
    dinv = jax.lax.rsqrt(deg)

    vx = jax.lax.dot_general(
        xs, vt_ref[...], (((2,), (0,)), ((), ())),
        preferred_element_type=jnp.float32,
    ) + vb_ref[...]
    agg = jax.lax.dot_general(
        adj, vx * dinv, (((2,), (1,)), ((0,), (0,))),
        preferred_element_type=jnp.float32,
    ) * dinv
    ux = jax.lax.dot_general(
        xs, ut_ref[...], (((2,), (0,)), ((), ())),
        preferred_element_type=jnp.float32,
    ) + ub_ref[...]
    o_ref[...] = jnp.maximum(xs + agg + ux, 0.0)


@jax.jit
def kernel(x, U_w, U_b, V_w, V_b):
    b, t, j, c = x.shape
    x3 = x.reshape(b, t, j * c)
    ut = U_w.T
    vt = V_w.T
    ub = U_b.reshape(1, c)
    vb = V_b.reshape(1, c)

    out3 = pl.pallas_call(
        functools.partial(_gcn_body, t=t, nb=NB),
        grid=(b // NB, j),
        in_specs=[
            pl.BlockSpec((NB, t, c), lambda bi, ji: (bi, 0, ji)),
            pl.BlockSpec((c, c), lambda bi, ji: (0, 0)),
            pl.BlockSpec((c, c), lambda bi, ji: (0, 0)),
            pl.BlockSpec((1, c), lambda bi, ji: (0, 0)),
            pl.BlockSpec((1, c), lambda bi, ji: (0, 0)),
        ],
        out_specs=pl.BlockSpec((NB, t, c), lambda bi, ji: (bi, 0, ji)),
        out_shape=jax.ShapeDtypeStruct((b, t, j * c), jnp.float32),
        compiler_params=pltpu.CompilerParams(
            dimension_semantics=("arbitrary", "arbitrary"),
        ),
    )(x3, ut, vt, ub, vb)
    return out3.reshape(b, t, j, c)


# speedup vs baseline: 4.4519x; 1.0465x over previous
import functools

import jax
import jax.numpy as jnp
from jax.experimental import pallas as pl
from jax.experimental.pallas import tpu as pltpu

NEIGHBOURS = 8
NB = 4


def _gcn_body(x_ref, ut_ref, vt_ref, ub_ref, vb_ref, o_ref, *, t, nb):
    xg = [x_ref[i] for i in range(nb)]

    sims = [
        jax.lax.dot_general(
            xb, xb, (((1,), (1,)), ((), ())),
            preferred_element_type=jnp.float32,
        )
        for xb in xg
    ]
    sim3 = jnp.stack(sims, axis=0)

    neg = jnp.float32(-jnp.inf)
    work = sim3
    thr = None
    for it in range(NEIGHBOURS):
        thr = jnp.max(work, axis=2, keepdims=True)
        if it < NEIGHBOURS - 1:
            work = jnp.where(work >= thr, neg, work)

    ones = jnp.ones((t, 128), jnp.float32)
    vt = vt_ref[...]
    ut = ut_ref[...]
    vb = vb_ref[...]
    ub = ub_ref[...]
    for i in range(nb):
        adj = (sims[i] >= thr[i]).astype(jnp.float32)
        deg = jnp.dot(adj, ones, preferred_element_type=jnp.float32)[:, 0:1]
        dinv = jax.lax.rsqrt(deg)
        vx = jnp.dot(xg[i], vt, preferred_element_type=jnp.float32) + vb
        agg = (
            jnp.dot(adj, vx * dinv, preferred_element_type=jnp.float32)
            * dinv
        )
        ux = jnp.dot(xg[i], ut, preferred_element_type=jnp.float32) + ub
        o_ref[i] = jnp.maximum(xg[i] + agg + ux, 0.0)


@jax.jit
def kernel(x, U_w, U_b, V_w, V_b):
    b, t, j, c = x.shape
    x3 = x.reshape(b, t, j * c)
    ut = U_w.T
    vt = V_w.T
    ub = U_b.reshape(1, c)
    vb = V_b.reshape(1, c)

    out3 = pl.pallas_call(
        functools.partial(_gcn_body, t=t, nb=NB),
        grid=(b // NB, j),
        in_specs=[
            pl.BlockSpec((NB, t, c), lambda bi, ji: (bi, 0, ji)),
            pl.BlockSpec((c, c), lambda bi, ji: (0, 0)),
            pl.BlockSpec((c, c), lambda bi, ji: (0, 0)),
            pl.BlockSpec((1, c), lambda bi, ji: (0, 0)),
            pl.BlockSpec((1, c), lambda bi, ji: (0, 0)),
        ],
        out_specs=pl.BlockSpec((NB, t, c), lambda bi, ji: (bi, 0, ji)),
        out_shape=jax.ShapeDtypeStruct((b, t, j * c), jnp.float32),
        compiler_params=pltpu.CompilerParams(
            dimension_semantics=("arbitrary", "arbitrary"),
        ),
    )(x3, ut, vt, ub, vb)
    return out3.reshape(b, t, j, c)


# speedup vs baseline: 4.5381x; 1.0194x over previous
import functools

import jax
import jax.numpy as jnp
from jax.experimental import pallas as pl
from jax.experimental.pallas import tpu as pltpu

NEIGHBOURS = 8
NB = 4


def _gcn_body(x_ref, ut_ref, vt_ref, ub_ref, vb_ref, o_ref, *, t, nb):
    xg = [x_ref[i] for i in range(nb)]

    sims = [
        jax.lax.dot_general(
            xb, xb, (((1,), (1,)), ((), ())),
            preferred_element_type=jnp.float32,
        )
        for xb in xg
    ]
    sim3 = jnp.stack(sims, axis=0)

    neg = jnp.float32(-jnp.inf)
    work = sim3
    thr = None
    for it in range(NEIGHBOURS):
        thr = jnp.max(work, axis=1, keepdims=True)
        if it < NEIGHBOURS - 1:
            work = jnp.where(work >= thr, neg, work)

    ones = jnp.ones((t, 128), jnp.float32)
    vt = vt_ref[...]
    ut = ut_ref[...]
    vb = vb_ref[...]
    ub = ub_ref[...]
    for i in range(nb):
        thr_col = jnp.transpose(thr[i], (1, 0))
        adj = (sims[i] >= thr_col).astype(jnp.float32)
        deg = jnp.dot(adj, ones, preferred_element_type=jnp.float32)[:, 0:1]
        dinv = jax.lax.rsqrt(deg)
        vx = jnp.dot(xg[i], vt, preferred_element_type=jnp.float32) + vb
        agg = (
            jnp.dot(adj, vx * dinv, preferred_element_type=jnp.float32)
            * dinv
        )
        ux = jnp.dot(xg[i], ut, preferred_element_type=jnp.float32) + ub
        o_ref[i] = jnp.maximum(xg[i] + agg + ux, 0.0)


@jax.jit
def kernel(x, U_w, U_b, V_w, V_b):
    b, t, j, c = x.shape
    x3 = x.reshape(b, t, j * c)
    ut = U_w.T
    vt = V_w.T
    ub = U_b.reshape(1, c)
    vb = V_b.reshape(1, c)

    out3 = pl.pallas_call(
        functools.partial(_gcn_body, t=t, nb=NB),
        grid=(b // NB, j),
        in_specs=[
            pl.BlockSpec((NB, t, c), lambda bi, ji: (bi, 0, ji)),
            pl.BlockSpec((c, c), lambda bi, ji: (0, 0)),
            pl.BlockSpec((c, c), lambda bi, ji: (0, 0)),
            pl.BlockSpec((1, c), lambda bi, ji: (0, 0)),
            pl.BlockSpec((1, c), lambda bi, ji: (0, 0)),
        ],
        out_specs=pl.BlockSpec((NB, t, c), lambda bi, ji: (bi, 0, ji)),
        out_shape=jax.ShapeDtypeStruct((b, t, j * c), jnp.float32),
        compiler_params=pltpu.CompilerParams(
            dimension_semantics=("arbitrary", "arbitrary"),
        ),
    )(x3, ut, vt, ub, vb)
    return out3.reshape(b, t, j, c)


# speedup vs baseline: 5.0301x; 1.1084x over previous
import functools

import jax
import jax.numpy as jnp
from jax.experimental import pallas as pl
from jax.experimental.pallas import tpu as pltpu

NEIGHBOURS = 8


def _gcn_body(x_ref, ut_ref, vt_ref, ub_ref, vb_ref, o_ref, *, t, nj):
    xb = x_ref[0]
    xg = [xb[:, jj, :] for jj in range(nj)]

    sims = [
        jax.lax.dot_general(
            xj, xj, (((1,), (1,)), ((), ())),
            preferred_element_type=jnp.float32,
        )
        for xj in xg
    ]
    sim3 = jnp.stack(sims, axis=0)

    neg = jnp.float32(-jnp.inf)
    work = sim3
    thr = None
    for it in range(NEIGHBOURS):
        thr = jnp.max(work, axis=1, keepdims=True)
        if it < NEIGHBOURS - 1:
            work = jnp.where(work >= thr, neg, work)

    ones = jnp.ones((t, 128), jnp.float32)
    vt = vt_ref[...]
    ut = ut_ref[...]
    vb = vb_ref[...]
    ub = ub_ref[...]
    outs = []
    for i in range(nj):
        thr_col = jnp.transpose(thr[i], (1, 0))
        adj = (sims[i] >= thr_col).astype(jnp.float32)
        deg = jnp.dot(adj, ones, preferred_element_type=jnp.float32)[:, 0:1]
        dinv = jax.lax.rsqrt(deg)
        vx = jnp.dot(xg[i], vt, preferred_element_type=jnp.float32) + vb
        agg = (
            jnp.dot(adj, vx * dinv, preferred_element_type=jnp.float32)
            * dinv
        )
        ux = jnp.dot(xg[i], ut, preferred_element_type=jnp.float32) + ub
        outs.append(jnp.maximum(xg[i] + agg + ux, 0.0))
    o_ref[0] = jnp.stack(outs, axis=1)


@jax.jit
def kernel(x, U_w, U_b, V_w, V_b):
    b, t, j, c = x.shape
    ut = U_w.T
    vt = V_w.T
    ub = U_b.reshape(1, c)
    vb = V_b.reshape(1, c)

    return pl.pallas_call(
        functools.partial(_gcn_body, t=t, nj=j),
        grid=(b,),
        in_specs=[
            pl.BlockSpec((1, t, j, c), lambda bi: (bi, 0, 0, 0)),
            pl.BlockSpec((c, c), lambda bi: (0, 0)),
            pl.BlockSpec((c, c), lambda bi: (0, 0)),
            pl.BlockSpec((1, c), lambda bi: (0, 0)),
            pl.BlockSpec((1, c), lambda bi: (0, 0)),
        ],
        out_specs=pl.BlockSpec((1, t, j, c), lambda bi: (bi, 0, 0, 0)),
        out_shape=jax.ShapeDtypeStruct((b, t, j, c), jnp.float32),
        compiler_params=pltpu.CompilerParams(
            dimension_semantics=("arbitrary",),
        ),
    )(x, ut, vt, ub, vb)
